# baseline (speedup 1.0000x reference)
import jax
import jax.numpy as jnp
from jax import lax
from jax.experimental import pallas as pl
from jax.experimental.pallas import tpu as pltpu
from jax.experimental.pallas import tpu_sc as plsc

NC, NS, L = 2, 16, 16
NW = NC * NS
B = 16384
F = 32
N = 1000000
NMAIN = 999936
BPW = B // NW
CHUNK = 128
NCH = BPW // CHUNK


W = 27776
NWIN = NMAIN // W
NBUF = 4


def _relayout_body(ut_hbm, it_hbm, utail_hbm, itail_hbm, ul_hbm, il_hbm,
                   buf0, buf1, buf2, buf3, lsem, ssem):
    wid = lax.axis_index("s") * NC + lax.axis_index("c")
    q = wid // 8
    k = wid % 8
    NT = N - NMAIN
    bufs = (buf0, buf1, buf2, buf3)
    srcs = (ut_hbm, it_hbm)
    dsts = (ul_hbm, il_hbm)
    TOT = 2 * NWIN

    def load(s):
        t, j = divmod(s, NWIN)
        return pltpu.async_copy(
            srcs[t].at[q, k, pl.ds(j * W, W)], bufs[s % NBUF], lsem)

    def store(s):
        t, j = divmod(s, NWIN)
        return pltpu.async_copy(
            bufs[s % NBUF], dsts[t].at[pl.ds(wid * N + j * W, W)], ssem)

    loads = [None] * TOT
    stores = [None] * TOT
    for s in range(min(NBUF - 1, TOT)):
        loads[s] = load(s)
    for s in range(TOT):
        loads[s].wait()
        stores[s] = store(s)
        nxt = s + NBUF - 1
        if nxt < TOT:
            if s >= 1:
                stores[s - 1].wait()
            loads[nxt] = load(nxt)
    for s in range(max(0, TOT - NBUF), TOT):
        stores[s].wait()

    for t in range(2):
        tail = (utail_hbm, itail_hbm)[t]
        pltpu.sync_copy(tail.at[pl.ds(wid * NT, NT)],
                        bufs[t].at[pl.ds(0, NT)])
        pltpu.sync_copy(bufs[t].at[pl.ds(0, NT)],
                        dsts[t].at[pl.ds(wid * N + NMAIN, NT)])


def _gather_body(user_hbm, item_hbm, ul_hbm, il_hbm, out_hbm,
                 uidx_v, iidx_v, gidx_v, ug_v, ig_v, out_v, sem):
    wid = lax.axis_index("s") * NC + lax.axis_index("c")
    base = wid * BPW

    pltpu.sync_copy(user_hbm.at[pl.ds(base, BPW)], uidx_v)
    pltpu.sync_copy(item_hbm.at[pl.ds(base, BPW)], iidx_v)

    def build(t, idx_v):
        def fac(c, carry):
            def vec(i, carry2):
                r = idx_v[pl.ds(i * L, L)]
                gidx_v[t, c, pl.ds(i * L, L)] = r + c * N
                return carry2
            return lax.fori_loop(0, BPW // L, vec, carry)
        lax.fori_loop(0, F, fac, 0)

    build(0, uidx_v)
    build(1, iidx_v)

    def zero(g, carry):
        out_v[pl.ds(g * L, L)] = jnp.zeros((L,), jnp.float32)
        return carry
    lax.fori_loop(0, BPW // L, zero, 0)

    copies = []
    for c in range(F):
        for k in range(NCH):
            sl = pl.ds(k * CHUNK, CHUNK)
            copies.append(pltpu.async_copy(
                ul_hbm.at[gidx_v.at[0, c, sl]], ug_v.at[c, sl], sem))
            copies.append(pltpu.async_copy(
                il_hbm.at[gidx_v.at[1, c, sl]], ig_v.at[c, sl], sem))

    for c in range(F):
        for cp in copies[c * 2 * NCH:(c + 1) * 2 * NCH]:
            cp.wait()

        def accum(g, carry, c=c):
            b0 = g * L
            sl = pl.ds(b0, L)
            out_v[sl] = out_v[sl] + ug_v[c, sl] * ig_v[c, sl]
            return carry
        lax.fori_loop(0, BPW // L, accum, 0)

    pltpu.sync_copy(out_v, out_hbm.at[pl.ds(base, BPW)])


def kernel(user, item, user_table, item_table):
    u3 = user_table.T.reshape(F // 8, 8, N)
    i3 = item_table.T.reshape(F // 8, 8, N)
    utail = user_table[NMAIN:].T.reshape(-1)
    itail = item_table[NMAIN:].T.reshape(-1)
    mesh = plsc.VectorSubcoreMesh(core_axis_name="c", subcore_axis_name="s")

    relayout = pl.kernel(
        _relayout_body,
        out_type=(jax.ShapeDtypeStruct((F * N,), jnp.float32),
                  jax.ShapeDtypeStruct((F * N,), jnp.float32)),
        mesh=mesh,
        compiler_params=pltpu.CompilerParams(
            needs_layout_passes=False, use_tc_tiling_on_sc=True),
        scratch_types=[
            pltpu.VMEM((W,), jnp.float32),
            pltpu.VMEM((W,), jnp.float32),
            pltpu.VMEM((W,), jnp.float32),
            pltpu.VMEM((W,), jnp.float32),
            pltpu.SemaphoreType.DMA,
            pltpu.SemaphoreType.DMA,
        ],
    )
    ul, il = relayout(u3, i3, utail, itail)

    gather = pl.kernel(
        _gather_body,
        out_type=jax.ShapeDtypeStruct((B,), jnp.float32),
        mesh=mesh,
        compiler_params=pltpu.CompilerParams(
            needs_layout_passes=False, use_tc_tiling_on_sc=False),
        scratch_types=[
            pltpu.VMEM((BPW,), jnp.int32),
            pltpu.VMEM((BPW,), jnp.int32),
            pltpu.VMEM((2, F, BPW), jnp.int32),
            pltpu.VMEM((F, BPW), jnp.float32),
            pltpu.VMEM((F, BPW), jnp.float32),
            pltpu.VMEM((BPW,), jnp.float32),
            pltpu.SemaphoreType.DMA,
        ],
    )
    return gather(user, item, ul, il)


# speedup vs baseline: 1.0127x; 1.0127x over previous
import jax
import jax.numpy as jnp
from jax import lax
from jax.experimental import pallas as pl
from jax.experimental.pallas import tpu as pltpu
from jax.experimental.pallas import tpu_sc as plsc

NC, NS, L = 2, 16, 16
NW = NC * NS
B = 16384
F = 32
N = 1000000
NMAIN = 999936
BPW = B // NW
CHUNK = 128
NCH = BPW // CHUNK


W = 35712
NWIN = NMAIN // W


def _relayout_body(ut_hbm, it_hbm, utail_hbm, itail_hbm, ul_hbm, il_hbm,
                   buf0, buf1, sem):
    wid = lax.axis_index("s") * NC + lax.axis_index("c")
    q = wid // 8
    k = wid % 8
    NT = N - NMAIN
    for src, tail, dst in ((ut_hbm, utail_hbm, ul_hbm),
                           (it_hbm, itail_hbm, il_hbm)):
        bufs = (buf0, buf1)
        loads = [None, None]
        loads[0] = pltpu.async_copy(src.at[q, k, pl.ds(0, W)], buf0, sem)
        for j in range(NWIN):
            nxt = (j + 1) % 2
            if j + 1 < NWIN:
                loads[nxt] = pltpu.async_copy(
                    src.at[q, k, pl.ds((j + 1) * W, W)], bufs[nxt], sem)
            loads[j % 2].wait()
            pltpu.sync_copy(bufs[j % 2], dst.at[pl.ds(wid * N + j * W, W)])
        pltpu.sync_copy(tail.at[pl.ds(wid * NT, NT)],
                        buf0.at[pl.ds(0, NT)])
        pltpu.sync_copy(buf0.at[pl.ds(0, NT)],
                        dst.at[pl.ds(wid * N + NMAIN, NT)])


def _gather_body(user_hbm, item_hbm, ul_hbm, il_hbm, out_hbm,
                 uidx_v, iidx_v, gidx_v, ug_v, ig_v, out_v, sem):
    wid = lax.axis_index("s") * NC + lax.axis_index("c")
    base = wid * BPW

    pltpu.sync_copy(user_hbm.at[pl.ds(base, BPW)], uidx_v)
    pltpu.sync_copy(item_hbm.at[pl.ds(base, BPW)], iidx_v)

    def build(t, idx_v):
        def fac(c, carry):
            def vec(i, carry2):
                r = idx_v[pl.ds(i * L, L)]
                gidx_v[t, c, pl.ds(i * L, L)] = r + c * N
                return carry2
            return lax.fori_loop(0, BPW // L, vec, carry)
        lax.fori_loop(0, F, fac, 0)

    build(0, uidx_v)
    build(1, iidx_v)

    copies = []
    for c in range(F):
        for k in range(NCH):
            sl = pl.ds(k * CHUNK, CHUNK)
            copies.append(pltpu.async_copy(
                ul_hbm.at[gidx_v.at[0, c, sl]], ug_v.at[c, sl], sem))
            copies.append(pltpu.async_copy(
                il_hbm.at[gidx_v.at[1, c, sl]], ig_v.at[c, sl], sem))
    for cp in copies:
        cp.wait()

    def group(g, carry):
        b0 = g * L
        acc = jnp.zeros((L,), jnp.float32)
        for c in range(F):
            acc = acc + ug_v[c, pl.ds(b0, L)] * ig_v[c, pl.ds(b0, L)]
        out_v[pl.ds(b0, L)] = acc
        return carry

    lax.fori_loop(0, BPW // L, group, 0)

    pltpu.sync_copy(out_v, out_hbm.at[pl.ds(base, BPW)])


def kernel(user, item, user_table, item_table):
    u3 = user_table.T.reshape(F // 8, 8, N)
    i3 = item_table.T.reshape(F // 8, 8, N)
    utail = user_table[NMAIN:].T.reshape(-1)
    itail = item_table[NMAIN:].T.reshape(-1)
    mesh = plsc.VectorSubcoreMesh(core_axis_name="c", subcore_axis_name="s")

    relayout = pl.kernel(
        _relayout_body,
        out_type=(jax.ShapeDtypeStruct((F * N,), jnp.float32),
                  jax.ShapeDtypeStruct((F * N,), jnp.float32)),
        mesh=mesh,
        compiler_params=pltpu.CompilerParams(
            needs_layout_passes=False, use_tc_tiling_on_sc=True),
        scratch_types=[
            pltpu.VMEM((W,), jnp.float32),
            pltpu.VMEM((W,), jnp.float32),
            pltpu.SemaphoreType.DMA,
        ],
    )
    ul, il = relayout(u3, i3, utail, itail)

    gather = pl.kernel(
        _gather_body,
        out_type=jax.ShapeDtypeStruct((B,), jnp.float32),
        mesh=mesh,
        compiler_params=pltpu.CompilerParams(
            needs_layout_passes=False, use_tc_tiling_on_sc=False),
        scratch_types=[
            pltpu.VMEM((BPW,), jnp.int32),
            pltpu.VMEM((BPW,), jnp.int32),
            pltpu.VMEM((2, F, BPW), jnp.int32),
            pltpu.VMEM((F, BPW), jnp.float32),
            pltpu.VMEM((F, BPW), jnp.float32),
            pltpu.VMEM((BPW,), jnp.float32),
            pltpu.SemaphoreType.DMA,
        ],
    )
    return gather(user, item, ul, il)
